# baseline (speedup 1.0000x reference)
import jax
import jax.numpy as jnp
from jax import lax
from jax.experimental import pallas as pl
from jax.experimental.pallas import tpu as pltpu
from jax.experimental.pallas import tpu_sc as plsc

N = 10000
E = 320000
D = 128
R = 20
NB = 10
EPS = 1e-12

NPAD = 10240
NSEG = NPAD * R
DW = 136
DBLK = 320
NBLKD = NPAD // DBLK
BSEG = DBLK * R
ACCR = BSEG + 8

NTILE = 16
EPT = 20480
EHLF = EPT
EPAD = EPT * NTILE
ECH = 2048
NCH = EHLF // ECH
LPAD = EHLF + 128

NBLK = 256
NGRID = NPAD // NBLK


def _sc_embed_body(xt, syn, pos, sense, lem, hraw, idxl, trows, hacc, sem):
  cid = lax.axis_index("c")
  sid = lax.axis_index("s")
  wid = sid * 2 + cid
  n0 = wid * 320
  for c in range(4):
    pltpu.sync_copy(xt.at[pl.ds(c * NPAD + n0, 320)],
                    idxl.at[pl.ds(c * 320, 320)])

  tables = (syn, pos, sense, lem)
  for b, bsz in ((0, 128), (128, 128), (256, 64)):
    pltpu.async_copy(tables[0].at[idxl.at[pl.ds(b, bsz)]],
                     hacc.at[pl.ds(b, bsz)], sem).wait()
    for t in (1, 2, 3):
      pltpu.async_copy(tables[t].at[idxl.at[pl.ds(t * 320 + b, bsz)]],
                       trows.at[pl.ds(0, bsz)], sem).wait()

      def add_loop(k, _):
        r = k // 8
        off = (k % 8) * 16
        plsc.addupdate(hacc.at[b + r, pl.ds(off, 16)],
                       trows[r, pl.ds(off, 16)])
        return _
      lax.fori_loop(0, bsz * 8, add_loop, 0)
  pltpu.sync_copy(hacc, hraw.at[pl.ds(n0, 320)])


def _sc_edges_body(srcp, dstp, relp, hp, s_out,
                   ebuf, clist, sbatch, gbatch, grows,
                   acc, gsem, ssem, esem):
  cid = lax.axis_index("c")
  sid = lax.axis_index("s")
  e0 = sid * EPT
  r0 = sid * (BSEG // NTILE)

  def fetch_edges(half, ch, par):
    base = e0 + half * EHLF + ch * ECH
    o = par * 3 * ECH
    pltpu.async_copy(srcp.at[pl.ds(base, ECH)],
                     ebuf.at[pl.ds(o, ECH)], esem)
    pltpu.async_copy(dstp.at[pl.ds(base, ECH)],
                     ebuf.at[pl.ds(o + ECH, ECH)], esem)
    pltpu.async_copy(relp.at[pl.ds(base, ECH)],
                     ebuf.at[pl.ds(o + 2 * ECH, ECH)], esem)

  def wait_edges(par):
    o = par * 3 * ECH
    for j in range(3):
      pltpu.make_async_copy(srcp.at[pl.ds(0, ECH)],
                            ebuf.at[pl.ds(o + j * ECH, ECH)], esem).wait()

  def zero_grows():
    zv = jnp.zeros((16,), jnp.float32)

    for rr in range(128):
      for w in range(8):
        grows[0, rr, pl.ds(w * 16, 16)] = zv
      grows[0, rr, pl.ds(DW - 16, 16)] = zv

  def one_pass(p, _):
    blk = cid * (NBLKD // 2) + p
    lo = blk * BSEG

    zero_grows()
    for j in range(BSEG // NTILE // 128):
      pltpu.sync_copy(grows.at[0], acc.at[pl.ds(r0 + j * 128, 128)])
    pltpu.sync_copy(grows.at[0, pl.ds(0, BSEG // NTILE % 128)],
                    acc.at[pl.ds(r0 + BSEG // NTILE // 128 * 128,
                                 BSEG // NTILE % 128)])
    plsc.subcore_barrier()

    def one_half(half, __):
      fetch_edges(half, 0, 0)

      def one_chunk(ch, wp):
        par = ch % 2
        wait_edges(par)

        @pl.when(ch + 1 < NCH)
        def _prefetch():
          fetch_edges(half, ch + 1, 1 - par)
        o = par * 3 * ECH

        def select(g, wpi):
          segs, ms, pcs, pks = [], [], [], []
          for u in range(2):
            q = o + g * 32 + u * 16
            s16 = ebuf[pl.ds(q, 16)]
            d16 = ebuf[pl.ds(ECH + q, 16)]
            r16 = ebuf[pl.ds(2 * ECH + q, 16)]
            seg = d16 * R + r16
            m = jnp.logical_and(seg >= lo, seg < lo + BSEG)
            ms.append(m)
            pcs.append(plsc.cumsum(jnp.where(m, 1, 0)))
            pks.append(lax.shift_left(seg, 14) | s16)
          plsc.store_scatter(clist, [wpi + pcs[0] - 1], pks[0], mask=ms[0])
          wp1 = wpi + jnp.max(pcs[0])
          plsc.store_scatter(clist, [wp1 + pcs[1] - 1], pks[1], mask=ms[1])
          return wp1 + jnp.max(pcs[1])
        return lax.fori_loop(0, ECH // 32, select, wp)
      nsel = lax.fori_loop(0, NCH, one_chunk, 0)

      pv = jnp.full((16,), (lo + BSEG) * 16384, jnp.int32)
      for k in range(8):
        clist[pl.ds(nsel + k * 16, 16)] = pv
      nbat = (nsel + 127) // 128

      def one_batch(j, _):
        db = j % 2
        for k in range(8):
          p16 = clist[pl.ds(j * 128 + k * 16, 16)]
          sbatch[db, pl.ds(k * 16, 16)] = (
              lax.shift_right_logical(p16, 14) - lo)
          gbatch[pl.ds(k * 16, 16)] = p16 & 16383

        @pl.when(j > 0)
        def _wait_prev():
          pltpu.make_async_copy(grows.at[0], acc.at[sbatch.at[0]],
                                ssem).wait()
        pltpu.async_copy(hp.at[gbatch], grows.at[db], gsem).wait()
        pltpu.async_copy(grows.at[db], acc.at[sbatch.at[db]], ssem, add=True)
        return _
      lax.fori_loop(0, nbat, one_batch, 0)

      @pl.when(nbat > 0)
      def _drain():
        pltpu.make_async_copy(grows.at[0], acc.at[sbatch.at[0]], ssem).wait()
      return __
    lax.fori_loop(0, 1, one_half, 0)

    plsc.subcore_barrier()
    pltpu.sync_copy(acc.at[pl.ds(r0, BSEG // NTILE)],
                    s_out.at[pl.ds(lo + r0, BSEG // NTILE)])
    plsc.subcore_barrier()
    return _
  lax.fori_loop(0, NBLKD // 2, one_pass, 0)


def _tc_prep_body(hraw_ref, gamma_ref, beta_ref, root_ref, bias_ref,
                  h_ref, hr_ref):
  xb = hraw_ref[...]
  mu = jnp.mean(xb, axis=-1, keepdims=True)
  var = jnp.mean(jnp.square(xb - mu), axis=-1, keepdims=True)
  h = (xb - mu) * lax.rsqrt(var + EPS) * gamma_ref[...] + beta_ref[...]
  h_ref[...] = h
  hr_ref[...] = jnp.dot(h, root_ref[...],
                        preferred_element_type=jnp.float32) + bias_ref[...]


def _tc_wperm_body(comp_ref, basis_ref, w_ref):
  w_ref[...] = jnp.dot(comp_ref[...], basis_ref[...],
                       preferred_element_type=jnp.float32)


def _tc_final_body(s_ref, w_ref, hr_ref, out_ref):
  acc = hr_ref[...]
  for r in range(R):
    sl = s_ref[:, r * DW:r * DW + D]
    cnt = s_ref[:, r * DW + D:r * DW + D + 1]
    acc = acc + jnp.dot(sl * (1.0 / jnp.maximum(cnt, 1.0)), w_ref[r],
                        preferred_element_type=jnp.float32)
  out_ref[...] = acc


def kernel(x, edge_index, e_id, edge_attrs, syn_emb, pos_emb, sense_emb,
           lem_emb, ln_gamma, ln_beta, comp, basis, root, bias):
  del e_id
  f32 = jnp.float32
  mesh = plsc.VectorSubcoreMesh(core_axis_name="c", subcore_axis_name="s",
                                num_cores=2, num_subcores=NTILE)

  xp = jnp.zeros((NPAD, 4), jnp.int32).at[:N].set(x.astype(jnp.int32))
  src = edge_index[0].astype(jnp.int32)
  dst = edge_index[1].astype(jnp.int32)
  rel = edge_attrs.astype(jnp.int32)
  pad = EPAD - E
  srcp = jnp.concatenate([src, jnp.zeros((pad,), jnp.int32)])
  dstp = jnp.concatenate([dst, jnp.full((pad,), NPAD, jnp.int32)])
  relp = jnp.concatenate([rel, jnp.zeros((pad,), jnp.int32)])


  sc_embed = pl.kernel(
      _sc_embed_body,
      out_type=jax.ShapeDtypeStruct((NPAD, D), f32),
      mesh=mesh,
      scratch_types=[
          pltpu.VMEM((1280,), jnp.int32),
          pltpu.VMEM((128, D), f32),
          pltpu.VMEM((320, D), f32),
          pltpu.SemaphoreType.DMA,
      ],
  )
  hraw = sc_embed(xp.T.reshape(-1), syn_emb, pos_emb, sense_emb, lem_emb)

  h, hr = pl.pallas_call(
      _tc_prep_body,
      grid=(NGRID,),
      in_specs=[
          pl.BlockSpec((NBLK, D), lambda i: (i, 0)),
          pl.BlockSpec((1, D), lambda i: (0, 0)),
          pl.BlockSpec((1, D), lambda i: (0, 0)),
          pl.BlockSpec((D, D), lambda i: (0, 0)),
          pl.BlockSpec((1, D), lambda i: (0, 0)),
      ],
      out_specs=[
          pl.BlockSpec((NBLK, D), lambda i: (i, 0)),
          pl.BlockSpec((NBLK, D), lambda i: (i, 0)),
      ],
      out_shape=[
          jax.ShapeDtypeStruct((NPAD, D), f32),
          jax.ShapeDtypeStruct((NPAD, D), f32),
      ],
  )(hraw, ln_gamma.reshape(1, D), ln_beta.reshape(1, D), root,
    bias.reshape(1, D))

  hp = jnp.concatenate(
      [h, jnp.ones((NPAD, 1), f32), jnp.zeros((NPAD, DW - D - 1), f32)],
      axis=1)

  wflat = pl.pallas_call(
      _tc_wperm_body,
      grid=(1,),
      in_specs=[
          pl.BlockSpec((R, NB), lambda i: (0, 0)),
          pl.BlockSpec((NB, D * D), lambda i: (0, 0)),
      ],
      out_specs=pl.BlockSpec((R, D * D), lambda i: (0, 0)),
      out_shape=jax.ShapeDtypeStruct((R, D * D), f32),
  )(comp, basis.reshape(NB, D * D))
  weight = wflat.reshape(R, D, D)

  sc_edges = pl.kernel(
      _sc_edges_body,
      compiler_params=pltpu.CompilerParams(use_tc_tiling_on_sc=False,
                                           needs_layout_passes=False),
      out_type=jax.ShapeDtypeStruct((NSEG, DW), f32),
      mesh=mesh,
      scratch_types=[
          pltpu.VMEM((2 * 3 * ECH,), jnp.int32),
          pltpu.VMEM((LPAD,), jnp.int32),
          pltpu.VMEM((2, 128), jnp.int32),
          pltpu.VMEM((128,), jnp.int32),
          pltpu.VMEM((2, 128, DW), f32),
          pltpu.VMEM_SHARED((ACCR, DW), f32),
          pltpu.SemaphoreType.DMA,
          pltpu.SemaphoreType.DMA,
          pltpu.SemaphoreType.DMA,
      ],
  )
  s_sum = sc_edges(srcp, dstp, relp, hp)

  out = pl.pallas_call(
      _tc_final_body,
      grid=(NGRID,),
      in_specs=[
          pl.BlockSpec((NBLK, R * DW), lambda i: (i, 0)),
          pl.BlockSpec((R, D, D), lambda i: (0, 0, 0)),
          pl.BlockSpec((NBLK, D), lambda i: (i, 0)),
      ],
      out_specs=pl.BlockSpec((NBLK, D), lambda i: (i, 0)),
      out_shape=jax.ShapeDtypeStruct((NPAD, D), f32),
  )(s_sum.reshape(NPAD, R * DW), weight, hr)

  return out[:N]


# speedup vs baseline: 1.0069x; 1.0069x over previous
import jax
import jax.numpy as jnp
from jax import lax
from jax.experimental import pallas as pl
from jax.experimental.pallas import tpu as pltpu
from jax.experimental.pallas import tpu_sc as plsc

N = 10000
E = 320000
D = 128
R = 20
NB = 10
EPS = 1e-12

NPAD = 10240
NSEG = NPAD * R
DW = 136
DBLK = 320
NBLKD = NPAD // DBLK
BSEG = DBLK * R
ACCR = BSEG + 8

NTILE = 16
EPT = 20480
EHLF = EPT
EPAD = EPT * NTILE
ECH = 2048
NCH = EHLF // ECH
LPAD = EHLF + 128

NBLK = 256
NGRID = NPAD // NBLK


def _sc_embed_body(xt, syn, pos, sense, lem, hraw, idxl, trows, hacc, sem):
  cid = lax.axis_index("c")
  sid = lax.axis_index("s")
  wid = sid * 2 + cid
  n0 = wid * 320
  for c in range(4):
    pltpu.sync_copy(xt.at[pl.ds(c * NPAD + n0, 320)],
                    idxl.at[pl.ds(c * 320, 320)])

  tables = (syn, pos, sense, lem)
  for b, bsz in ((0, 128), (128, 128), (256, 64)):
    pltpu.async_copy(tables[0].at[idxl.at[pl.ds(b, bsz)]],
                     hacc.at[pl.ds(b, bsz)], sem).wait()
    for t in (1, 2, 3):
      pltpu.async_copy(tables[t].at[idxl.at[pl.ds(t * 320 + b, bsz)]],
                       trows.at[pl.ds(0, bsz)], sem).wait()

      def add_loop(k, _):
        r = k // 8
        off = (k % 8) * 16
        plsc.addupdate(hacc.at[b + r, pl.ds(off, 16)],
                       trows[r, pl.ds(off, 16)])
        return _
      lax.fori_loop(0, bsz * 8, add_loop, 0)
  pltpu.sync_copy(hacc, hraw.at[pl.ds(n0, 320)])


def _sc_edges_body(srcp, dstp, relp, hp, s_out,
                   ebuf, clist, sbatch, gbatch, grows,
                   acc, gsem, ssem, esem):
  cid = lax.axis_index("c")
  sid = lax.axis_index("s")
  e0 = sid * EPT
  r0 = sid * (BSEG // NTILE)

  def fetch_edges(half, ch, par):
    base = e0 + half * EHLF + ch * ECH
    o = par * 3 * ECH
    pltpu.async_copy(srcp.at[pl.ds(base, ECH)],
                     ebuf.at[pl.ds(o, ECH)], esem)
    pltpu.async_copy(dstp.at[pl.ds(base, ECH)],
                     ebuf.at[pl.ds(o + ECH, ECH)], esem)
    pltpu.async_copy(relp.at[pl.ds(base, ECH)],
                     ebuf.at[pl.ds(o + 2 * ECH, ECH)], esem)

  def wait_edges(par):
    o = par * 3 * ECH
    for j in range(3):
      pltpu.make_async_copy(srcp.at[pl.ds(0, ECH)],
                            ebuf.at[pl.ds(o + j * ECH, ECH)], esem).wait()

  def zero_grows():
    zv = jnp.zeros((16,), jnp.float32)

    for rr in range(128):
      for w in range(8):
        grows[0, rr, pl.ds(w * 16, 16)] = zv
      grows[0, rr, pl.ds(DW - 16, 16)] = zv

  def one_pass(p, _):
    blk = cid * (NBLKD // 2) + p
    lo = blk * BSEG

    zero_grows()
    for j in range(BSEG // NTILE // 128):
      pltpu.sync_copy(grows.at[0], acc.at[pl.ds(r0 + j * 128, 128)])
    pltpu.sync_copy(grows.at[0, pl.ds(0, BSEG // NTILE % 128)],
                    acc.at[pl.ds(r0 + BSEG // NTILE // 128 * 128,
                                 BSEG // NTILE % 128)])
    plsc.subcore_barrier()

    def one_half(half, __):
      fetch_edges(half, 0, 0)

      def one_chunk(ch, wp):
        par = ch % 2
        wait_edges(par)

        @pl.when(ch + 1 < NCH)
        def _prefetch():
          fetch_edges(half, ch + 1, 1 - par)
        o = par * 3 * ECH

        def select(g, wpi):
          s16 = ebuf[pl.ds(o + g * 16, 16)]
          d16 = ebuf[pl.ds(o + ECH + g * 16, 16)]
          r16 = ebuf[pl.ds(o + 2 * ECH + g * 16, 16)]
          seg = d16 * R + r16
          m = jnp.logical_and(seg >= lo, seg < lo + BSEG)
          pc = plsc.cumsum(jnp.where(m, 1, 0))
          plsc.store_scatter(clist, [wpi + pc - 1],
                             lax.shift_left(seg, 14) | s16, mask=m)
          return wpi + jnp.max(pc)
        return lax.fori_loop(0, ECH // 16, select, wp)
      nsel = lax.fori_loop(0, NCH, one_chunk, 0)

      pv = jnp.full((16,), (lo + BSEG) * 16384, jnp.int32)
      for k in range(8):
        clist[pl.ds(nsel + k * 16, 16)] = pv
      nbat = (nsel + 127) // 128

      def one_batch(j, _):
        db = j % 2
        for k in range(8):
          p16 = clist[pl.ds(j * 128 + k * 16, 16)]
          sbatch[db, pl.ds(k * 16, 16)] = (
              lax.shift_right_logical(p16, 14) - lo)
          gbatch[pl.ds(k * 16, 16)] = p16 & 16383

        @pl.when(j > 0)
        def _wait_prev():
          pltpu.make_async_copy(grows.at[0], acc.at[sbatch.at[0]],
                                ssem).wait()
        pltpu.async_copy(hp.at[gbatch], grows.at[db], gsem).wait()
        pltpu.async_copy(grows.at[db], acc.at[sbatch.at[db]], ssem, add=True)
        return _
      lax.fori_loop(0, nbat, one_batch, 0)

      @pl.when(nbat > 0)
      def _drain():
        pltpu.make_async_copy(grows.at[0], acc.at[sbatch.at[0]], ssem).wait()
      return __
    lax.fori_loop(0, 1, one_half, 0)

    plsc.subcore_barrier()
    pltpu.sync_copy(acc.at[pl.ds(r0, BSEG // NTILE)],
                    s_out.at[pl.ds(lo + r0, BSEG // NTILE)])
    plsc.subcore_barrier()
    return _
  lax.fori_loop(0, NBLKD // 2, one_pass, 0)


def _tc_prep_body(hraw_ref, gamma_ref, beta_ref, root_ref, bias_ref,
                  h_ref, hr_ref):
  xb = hraw_ref[...]
  mu = jnp.mean(xb, axis=-1, keepdims=True)
  var = jnp.mean(jnp.square(xb - mu), axis=-1, keepdims=True)
  h = (xb - mu) * lax.rsqrt(var + EPS) * gamma_ref[...] + beta_ref[...]
  h_ref[...] = h
  hr_ref[...] = jnp.dot(h, root_ref[...],
                        preferred_element_type=jnp.float32) + bias_ref[...]


def _tc_wperm_body(comp_ref, basis_ref, w_ref):
  w_ref[...] = jnp.dot(comp_ref[...], basis_ref[...],
                       preferred_element_type=jnp.float32)


def _tc_final_body(s_ref, w_ref, hr_ref, out_ref):
  acc = hr_ref[...]
  for r in range(R):
    sl = s_ref[:, r * DW:r * DW + D]
    cnt = s_ref[:, r * DW + D:r * DW + D + 1]
    acc = acc + jnp.dot(sl * (1.0 / jnp.maximum(cnt, 1.0)), w_ref[r],
                        preferred_element_type=jnp.float32)
  out_ref[...] = acc


def kernel(x, edge_index, e_id, edge_attrs, syn_emb, pos_emb, sense_emb,
           lem_emb, ln_gamma, ln_beta, comp, basis, root, bias):
  del e_id
  f32 = jnp.float32
  mesh = plsc.VectorSubcoreMesh(core_axis_name="c", subcore_axis_name="s",
                                num_cores=2, num_subcores=NTILE)

  xp = jnp.zeros((NPAD, 4), jnp.int32).at[:N].set(x.astype(jnp.int32))
  src = edge_index[0].astype(jnp.int32)
  dst = edge_index[1].astype(jnp.int32)
  rel = edge_attrs.astype(jnp.int32)
  pad = EPAD - E
  srcp = jnp.concatenate([src, jnp.zeros((pad,), jnp.int32)])
  dstp = jnp.concatenate([dst, jnp.full((pad,), NPAD, jnp.int32)])
  relp = jnp.concatenate([rel, jnp.zeros((pad,), jnp.int32)])


  sc_embed = pl.kernel(
      _sc_embed_body,
      out_type=jax.ShapeDtypeStruct((NPAD, D), f32),
      mesh=mesh,
      scratch_types=[
          pltpu.VMEM((1280,), jnp.int32),
          pltpu.VMEM((128, D), f32),
          pltpu.VMEM((320, D), f32),
          pltpu.SemaphoreType.DMA,
      ],
  )
  hraw = sc_embed(xp.T.reshape(-1), syn_emb, pos_emb, sense_emb, lem_emb)

  h, hr = pl.pallas_call(
      _tc_prep_body,
      grid=(NGRID,),
      in_specs=[
          pl.BlockSpec((NBLK, D), lambda i: (i, 0)),
          pl.BlockSpec((1, D), lambda i: (0, 0)),
          pl.BlockSpec((1, D), lambda i: (0, 0)),
          pl.BlockSpec((D, D), lambda i: (0, 0)),
          pl.BlockSpec((1, D), lambda i: (0, 0)),
      ],
      out_specs=[
          pl.BlockSpec((NBLK, D), lambda i: (i, 0)),
          pl.BlockSpec((NBLK, D), lambda i: (i, 0)),
      ],
      out_shape=[
          jax.ShapeDtypeStruct((NPAD, D), f32),
          jax.ShapeDtypeStruct((NPAD, D), f32),
      ],
  )(hraw, ln_gamma.reshape(1, D), ln_beta.reshape(1, D), root,
    bias.reshape(1, D))

  hp = jnp.concatenate(
      [h, jnp.ones((NPAD, 1), f32), jnp.zeros((NPAD, DW - D - 1), f32)],
      axis=1)

  wflat = pl.pallas_call(
      _tc_wperm_body,
      grid=(1,),
      in_specs=[
          pl.BlockSpec((R, NB), lambda i: (0, 0)),
          pl.BlockSpec((NB, D * D), lambda i: (0, 0)),
      ],
      out_specs=pl.BlockSpec((R, D * D), lambda i: (0, 0)),
      out_shape=jax.ShapeDtypeStruct((R, D * D), f32),
  )(comp, basis.reshape(NB, D * D))
  weight = wflat.reshape(R, D, D)

  sc_edges = pl.kernel(
      _sc_edges_body,
      compiler_params=pltpu.CompilerParams(use_tc_tiling_on_sc=False,
                                           needs_layout_passes=False),
      out_type=jax.ShapeDtypeStruct((NSEG, DW), f32),
      mesh=mesh,
      scratch_types=[
          pltpu.VMEM((2 * 3 * ECH,), jnp.int32),
          pltpu.VMEM((LPAD,), jnp.int32),
          pltpu.VMEM((2, 128), jnp.int32),
          pltpu.VMEM((128,), jnp.int32),
          pltpu.VMEM((2, 128, DW), f32),
          pltpu.VMEM_SHARED((ACCR, DW), f32),
          pltpu.SemaphoreType.DMA,
          pltpu.SemaphoreType.DMA,
          pltpu.SemaphoreType.DMA,
      ],
  )
  s_sum = sc_edges(srcp, dstp, relp, hp)

  out = pl.pallas_call(
      _tc_final_body,
      grid=(NGRID,),
      in_specs=[
          pl.BlockSpec((NBLK, R * DW), lambda i: (i, 0)),
          pl.BlockSpec((R, D, D), lambda i: (0, 0, 0)),
          pl.BlockSpec((NBLK, D), lambda i: (i, 0)),
      ],
      out_specs=pl.BlockSpec((NBLK, D), lambda i: (i, 0)),
      out_shape=jax.ShapeDtypeStruct((NPAD, D), f32),
  )(s_sum.reshape(NPAD, R * DW), weight, hr)

  return out[:N]
